# baseline (speedup 1.0000x reference)
import functools

import jax
import jax.numpy as jnp
from jax import lax
from jax.experimental import pallas as pl
from jax.experimental.pallas import tpu as pltpu
from jax.experimental.pallas import tpu_sc as plsc

B = 8
K = 4096
C = 256
N = 512

_NC, _NS = 2, 16
_NW = _NC * _NS
_ROWS = B * C
_RPW = _ROWS // _NW
_TPB = _NW // B

_GC = 4
_NG = _RPW // _GC
_CHUNK = 128
_CPG = _GC * N // _CHUNK


def _fps_body(x_ref, y_ref, z_ref, inds_ref):
    x = x_ref[...]
    y = y_ref[...]
    z = z_ref[...]
    iota = lax.broadcasted_iota(jnp.int32, (B, K), 1)

    lx = x[:, 0:1]
    ly = y[:, 0:1]
    lz = z[:, 0:1]
    inds_ref[0:1, :] = jnp.zeros((1, B), jnp.int32)

    dists0 = jnp.full((B, K), 1e10, dtype=jnp.float32)

    def body(i, carry):
        dists, lx, ly, lz = carry
        dx = x - lx
        dy = y - ly
        dz = z - lz
        d = dx * dx + dy * dy + dz * dz
        dists = jnp.minimum(dists, d)
        m = jnp.max(dists, axis=1, keepdims=True)
        idx = jnp.min(jnp.where(dists == m, iota, K), axis=1, keepdims=True)
        sel = iota == idx
        lx = jnp.sum(jnp.where(sel, x, 0.0), axis=1, keepdims=True)
        ly = jnp.sum(jnp.where(sel, y, 0.0), axis=1, keepdims=True)
        lz = jnp.sum(jnp.where(sel, z, 0.0), axis=1, keepdims=True)
        inds_ref[pl.ds(i, 1), :] = idx.T
        return dists, lx, ly, lz

    lax.fori_loop(1, N, body, (dists0, lx, ly, lz))


@jax.jit
def _fps(x, y, z):
    return pl.pallas_call(
        _fps_body,
        out_shape=jax.ShapeDtypeStruct((N, B), jnp.int32),
    )(x, y, z)


def _gather_body(feat_hbm, xyz_hbm, inds_hbm,
                 out_hbm, nxyz_hbm, sinds_hbm,
                 inds_v, sidx, xidx, xtrip, xrow, idxA, idxB, rowA, rowB,
                 semA, semB):
    wid = lax.axis_index("s") * _NC + lax.axis_index("c")
    b = wid // _TPB
    q = wid % _TPB
    c0 = q * _RPW

    lane = lax.iota(jnp.int32, 16) * B + b
    for j in range(4):
        for s in range(8):
            sidx[j, pl.ds(s * 16, 16)] = lane + (j * _CHUNK + s * 16) * B
    for j in range(4):
        pltpu.async_copy(inds_hbm.at[sidx.at[j]],
                         inds_v.at[pl.ds(j * _CHUNK, _CHUNK)], semA)
    pltpu.make_async_copy(inds_hbm.at[pl.ds(0, N)], inds_v, semA).wait()

    pltpu.sync_copy(inds_v, sinds_hbm.at[pl.ds(b * N, N)])

    iota16 = lax.iota(jnp.int32, 16)
    for s in range(24):
        p = iota16 + s * 16
        jv = lax.shift_right_logical(p * 21846, 16)
        sidx[s // 8, pl.ds((s % 8) * 16, 16)] = (q * _CHUNK + jv) * B + b
    for j in range(3):
        pltpu.async_copy(inds_hbm.at[sidx.at[j]],
                         xtrip.at[pl.ds(j * _CHUNK, _CHUNK)], semB)
    pltpu.make_async_copy(inds_hbm.at[pl.ds(0, 3 * _CHUNK)], xtrip, semB).wait()
    for s in range(24):
        p = iota16 + s * 16
        jv = lax.shift_right_logical(p * 21846, 16)
        cv = p - jv * 3
        xidx[s // 8, pl.ds((s % 8) * 16, 16)] = (
            (xtrip[pl.ds(s * 16, 16)] + b * K) * 3 + cv)
    for j in range(3):
        pltpu.async_copy(xyz_hbm.at[xidx.at[j]],
                         xrow.at[pl.ds(j * _CHUNK, _CHUNK)], semB)
    pltpu.make_async_copy(xyz_hbm.at[pl.ds(0, 3 * _CHUNK)], xrow, semB).wait()
    pltpu.sync_copy(xrow, nxyz_hbm.at[pl.ds((b * N + q * _CHUNK) * 3, 3 * _CHUNK)])

    base0 = b * (C * K) + c0 * K

    def build(idxbuf, g):
        for j in range(_CPG):
            base = base0 + (g * _GC + j // 4) * K
            for s in range(8):
                idxbuf[j, pl.ds(s * 16, 16)] = (
                    inds_v[pl.ds((j % 4) * _CHUNK + s * 16, 16)] + base)

    def fire(idxbuf, rowbuf, sem):
        for j in range(_CPG):
            pltpu.async_copy(feat_hbm.at[idxbuf.at[j]],
                             rowbuf.at[pl.ds(j * _CHUNK, _CHUNK)], sem)

    def drain(rowbuf, sem):
        pltpu.make_async_copy(feat_hbm.at[pl.ds(0, _GC * N)], rowbuf, sem).wait()

    def out(rowbuf, g):
        pltpu.sync_copy(rowbuf,
                        out_hbm.at[pl.ds((b * C + c0 + g * _GC) * N, _GC * N)])

    build(idxA, 0)
    fire(idxA, rowA, semA)

    def body(it, carry):
        gA = 2 * it
        build(idxB, gA + 1)
        fire(idxB, rowB, semB)
        drain(rowA, semA)
        out(rowA, gA)

        @pl.when(it < _NG // 2 - 1)
        def _():
            build(idxA, gA + 2)
            fire(idxA, rowA, semA)

        drain(rowB, semB)
        out(rowB, gA + 1)
        return carry

    lax.fori_loop(0, _NG // 2, body, 0)


_sc_gather = pl.kernel(
    _gather_body,
    out_type=(
        jax.ShapeDtypeStruct((B * C * N,), jnp.float32),
        jax.ShapeDtypeStruct((B * N * 3,), jnp.float32),
        jax.ShapeDtypeStruct((B * N,), jnp.int32),
    ),
    mesh=plsc.VectorSubcoreMesh(core_axis_name="c", subcore_axis_name="s"),
    scratch_types=[
        pltpu.VMEM((N,), jnp.int32),
        pltpu.VMEM((4, _CHUNK), jnp.int32),
        pltpu.VMEM((3, _CHUNK), jnp.int32),
        pltpu.VMEM((3 * _CHUNK,), jnp.int32),
        pltpu.VMEM((3 * _CHUNK,), jnp.float32),
        pltpu.VMEM((_CPG, _CHUNK), jnp.int32),
        pltpu.VMEM((_CPG, _CHUNK), jnp.int32),
        pltpu.VMEM((_GC * N,), jnp.float32),
        pltpu.VMEM((_GC * N,), jnp.float32),
        pltpu.SemaphoreType.DMA,
        pltpu.SemaphoreType.DMA,
    ],
)


@jax.jit
def kernel(xyz, features):
    x = xyz[:, :, 0]
    y = xyz[:, :, 1]
    z = xyz[:, :, 2]
    inds_nm = _fps(x, y, z)
    out_flat, nxyz_flat, sinds_flat = _sc_gather(
        features.reshape(-1), xyz.reshape(-1), inds_nm.reshape(-1))
    new_features = out_flat.reshape(B, C, N)
    sample_inds = sinds_flat.reshape(B, N)
    new_xyz = nxyz_flat.reshape(B, N, 3)
    return new_xyz, new_features, sample_inds


# speedup vs baseline: 1.0217x; 1.0217x over previous
import functools

import jax
import jax.numpy as jnp
from jax import lax
from jax.experimental import pallas as pl
from jax.experimental.pallas import tpu as pltpu
from jax.experimental.pallas import tpu_sc as plsc

B = 8
K = 4096
C = 256
N = 512

_NC, _NS = 2, 16
_NW = _NC * _NS
_ROWS = B * C
_RPW = _ROWS // _NW
_TPB = _NW // B

_GC = 4
_NG = _RPW // _GC
_CHUNK = 128
_CPG = _GC * N // _CHUNK


def _fps_body(x_ref, y_ref, z_ref, inds_ref):
    x = x_ref[...]
    y = y_ref[...]
    z = z_ref[...]
    iota = lax.broadcasted_iota(jnp.int32, (B, K), 1)

    lx = x[:, 0:1]
    ly = y[:, 0:1]
    lz = z[:, 0:1]

    dists0 = jnp.full((B, K), 1e10, dtype=jnp.float32)
    acc0 = jnp.zeros((B, N), jnp.int32)
    lane_n = lax.broadcasted_iota(jnp.int32, (B, N), 1)

    def body(i, carry):
        dists, lx, ly, lz, acc = carry
        dx = x - lx
        dy = y - ly
        dz = z - lz
        d = dx * dx + dy * dy + dz * dz
        dists = jnp.minimum(dists, d)
        m = jnp.max(dists, axis=1, keepdims=True)
        idx = jnp.min(jnp.where(dists == m, iota, K), axis=1, keepdims=True)
        sel = iota == idx
        lx = jnp.sum(jnp.where(sel, x, 0.0), axis=1, keepdims=True)
        ly = jnp.sum(jnp.where(sel, y, 0.0), axis=1, keepdims=True)
        lz = jnp.sum(jnp.where(sel, z, 0.0), axis=1, keepdims=True)
        acc = jnp.where(lane_n == i, idx, acc)
        return dists, lx, ly, lz, acc

    carry = lax.fori_loop(1, N, body, (dists0, lx, ly, lz, acc0))
    inds_ref[...] = carry[4]


@jax.jit
def _fps(x, y, z):
    return pl.pallas_call(
        _fps_body,
        out_shape=jax.ShapeDtypeStruct((B, N), jnp.int32),
    )(x, y, z)


def _gather_body(feat_hbm, xyz_hbm, inds_hbm,
                 out_hbm, nxyz_hbm,
                 inds_v, sidx, xidx, xtrip, xrow, idxA, idxB, rowA, rowB,
                 semA, semB):
    wid = lax.axis_index("s") * _NC + lax.axis_index("c")
    b = wid // _TPB
    q = wid % _TPB
    c0 = q * _RPW

    pltpu.sync_copy(inds_hbm.at[pl.ds(b * N, N)], inds_v)

    iota16 = lax.iota(jnp.int32, 16)
    for s in range(24):
        p = iota16 + s * 16
        jv = lax.shift_right_logical(p * 21846, 16)
        sidx[s // 8, pl.ds((s % 8) * 16, 16)] = b * N + q * _CHUNK + jv
    for j in range(3):
        pltpu.async_copy(inds_hbm.at[sidx.at[j]],
                         xtrip.at[pl.ds(j * _CHUNK, _CHUNK)], semB)
    pltpu.make_async_copy(inds_hbm.at[pl.ds(0, 3 * _CHUNK)], xtrip, semB).wait()
    for s in range(24):
        p = iota16 + s * 16
        jv = lax.shift_right_logical(p * 21846, 16)
        cv = p - jv * 3
        xidx[s // 8, pl.ds((s % 8) * 16, 16)] = (
            (xtrip[pl.ds(s * 16, 16)] + b * K) * 3 + cv)
    for j in range(3):
        pltpu.async_copy(xyz_hbm.at[xidx.at[j]],
                         xrow.at[pl.ds(j * _CHUNK, _CHUNK)], semB)
    pltpu.make_async_copy(xyz_hbm.at[pl.ds(0, 3 * _CHUNK)], xrow, semB).wait()
    pltpu.sync_copy(xrow, nxyz_hbm.at[pl.ds((b * N + q * _CHUNK) * 3, 3 * _CHUNK)])

    base0 = b * (C * K) + c0 * K

    def build(idxbuf, g):
        for j in range(_CPG):
            base = base0 + (g * _GC + j // 4) * K
            for s in range(8):
                idxbuf[j, pl.ds(s * 16, 16)] = (
                    inds_v[pl.ds((j % 4) * _CHUNK + s * 16, 16)] + base)

    def fire(idxbuf, rowbuf, sem):
        for j in range(_CPG):
            pltpu.async_copy(feat_hbm.at[idxbuf.at[j]],
                             rowbuf.at[pl.ds(j * _CHUNK, _CHUNK)], sem)

    def drain(rowbuf, sem):
        pltpu.make_async_copy(feat_hbm.at[pl.ds(0, _GC * N)], rowbuf, sem).wait()

    def out(rowbuf, g):
        pltpu.sync_copy(rowbuf,
                        out_hbm.at[pl.ds((b * C + c0 + g * _GC) * N, _GC * N)])

    build(idxA, 0)
    fire(idxA, rowA, semA)

    def body(it, carry):
        gA = 2 * it
        build(idxB, gA + 1)
        fire(idxB, rowB, semB)
        drain(rowA, semA)
        out(rowA, gA)

        @pl.when(it < _NG // 2 - 1)
        def _():
            build(idxA, gA + 2)
            fire(idxA, rowA, semA)

        drain(rowB, semB)
        out(rowB, gA + 1)
        return carry

    lax.fori_loop(0, _NG // 2, body, 0)


_sc_gather = pl.kernel(
    _gather_body,
    out_type=(
        jax.ShapeDtypeStruct((B * C * N,), jnp.float32),
        jax.ShapeDtypeStruct((B * N * 3,), jnp.float32),
    ),
    mesh=plsc.VectorSubcoreMesh(core_axis_name="c", subcore_axis_name="s"),
    scratch_types=[
        pltpu.VMEM((N,), jnp.int32),
        pltpu.VMEM((4, _CHUNK), jnp.int32),
        pltpu.VMEM((3, _CHUNK), jnp.int32),
        pltpu.VMEM((3 * _CHUNK,), jnp.int32),
        pltpu.VMEM((3 * _CHUNK,), jnp.float32),
        pltpu.VMEM((_CPG, _CHUNK), jnp.int32),
        pltpu.VMEM((_CPG, _CHUNK), jnp.int32),
        pltpu.VMEM((_GC * N,), jnp.float32),
        pltpu.VMEM((_GC * N,), jnp.float32),
        pltpu.SemaphoreType.DMA,
        pltpu.SemaphoreType.DMA,
    ],
)


@jax.jit
def kernel(xyz, features):
    x = xyz[:, :, 0]
    y = xyz[:, :, 1]
    z = xyz[:, :, 2]
    sample_inds = _fps(x, y, z)
    out_flat, nxyz_flat = _sc_gather(
        features.reshape(-1), xyz.reshape(-1), sample_inds.reshape(-1))
    new_features = out_flat.reshape(B, C, N)
    new_xyz = nxyz_flat.reshape(B, N, 3)
    return new_xyz, new_features, sample_inds


# speedup vs baseline: 1.1429x; 1.1186x over previous
import functools

import jax
import jax.numpy as jnp
from jax import lax
from jax.experimental import pallas as pl
from jax.experimental.pallas import tpu as pltpu
from jax.experimental.pallas import tpu_sc as plsc

B = 8
K = 4096
C = 256
N = 512

_NC, _NS = 2, 16
_NW = _NC * _NS
_ROWS = B * C
_RPW = _ROWS // _NW
_TPB = _NW // B

_GC = 4
_NG = _RPW // _GC
_CHUNK = 128
_CPG = _GC * N // _CHUNK


def _fps_body(x_ref, y_ref, z_ref, inds_ref, nxyz_ref):
    x = x_ref[...]
    y = y_ref[...]
    z = z_ref[...]
    iota = lax.broadcasted_iota(jnp.int32, (B, K), 1)

    lx = x[:, 0:1]
    ly = y[:, 0:1]
    lz = z[:, 0:1]

    dists0 = jnp.full((B, K), 1e10, dtype=jnp.float32)
    lane_n = lax.broadcasted_iota(jnp.int32, (B, N), 1)
    acc0 = jnp.zeros((B, N), jnp.int32)
    first = lane_n == 0
    ax0 = jnp.where(first, lx, 0.0)
    ay0 = jnp.where(first, ly, 0.0)
    az0 = jnp.where(first, lz, 0.0)

    def body(i, carry):
        dists, lx, ly, lz, acc, ax, ay, az = carry
        dx = x - lx
        dy = y - ly
        dz = z - lz
        d = dx * dx + dy * dy + dz * dz
        dists = jnp.minimum(dists, d)
        m = jnp.max(dists, axis=1, keepdims=True)
        idx = jnp.min(jnp.where(dists == m, iota, K), axis=1, keepdims=True)
        sel = iota == idx
        lx = jnp.sum(jnp.where(sel, x, 0.0), axis=1, keepdims=True)
        ly = jnp.sum(jnp.where(sel, y, 0.0), axis=1, keepdims=True)
        lz = jnp.sum(jnp.where(sel, z, 0.0), axis=1, keepdims=True)
        hit = lane_n == i
        acc = jnp.where(hit, idx, acc)
        ax = jnp.where(hit, lx, ax)
        ay = jnp.where(hit, ly, ay)
        az = jnp.where(hit, lz, az)
        return dists, lx, ly, lz, acc, ax, ay, az

    carry = lax.fori_loop(1, N, body,
                          (dists0, lx, ly, lz, acc0, ax0, ay0, az0))
    inds_ref[...] = carry[4]
    nxyz_ref[...] = jnp.stack([carry[5], carry[6], carry[7]],
                              axis=-1).reshape(B, 3 * N)


@jax.jit
def _fps(x, y, z):
    return pl.pallas_call(
        _fps_body,
        out_shape=(
            jax.ShapeDtypeStruct((B, N), jnp.int32),
            jax.ShapeDtypeStruct((B, 3 * N), jnp.float32),
        ),
    )(x, y, z)


def _gather_body(feat_hbm, inds_hbm, out_hbm,
                 inds_v, idxA, idxB, rowA, rowB, semA, semB):
    wid = lax.axis_index("s") * _NC + lax.axis_index("c")
    b = wid // _TPB
    q = wid % _TPB
    c0 = q * _RPW

    pltpu.sync_copy(inds_hbm.at[pl.ds(b * N, N)], inds_v)

    base0 = b * (C * K) + c0 * K

    def build(idxbuf, g):
        for j in range(_CPG):
            base = base0 + (g * _GC + j // 4) * K
            for s in range(8):
                idxbuf[j, pl.ds(s * 16, 16)] = (
                    inds_v[pl.ds((j % 4) * _CHUNK + s * 16, 16)] + base)

    def fire(idxbuf, rowbuf, sem):
        for j in range(_CPG):
            pltpu.async_copy(feat_hbm.at[idxbuf.at[j]],
                             rowbuf.at[pl.ds(j * _CHUNK, _CHUNK)], sem)

    def drain(rowbuf, sem):
        pltpu.make_async_copy(feat_hbm.at[pl.ds(0, _GC * N)], rowbuf, sem).wait()

    def out(rowbuf, g):
        pltpu.sync_copy(rowbuf,
                        out_hbm.at[pl.ds((b * C + c0 + g * _GC) * N, _GC * N)])

    build(idxA, 0)
    fire(idxA, rowA, semA)

    def body(it, carry):
        gA = 2 * it
        build(idxB, gA + 1)
        fire(idxB, rowB, semB)
        drain(rowA, semA)
        out(rowA, gA)

        @pl.when(it < _NG // 2 - 1)
        def _():
            build(idxA, gA + 2)
            fire(idxA, rowA, semA)

        drain(rowB, semB)
        out(rowB, gA + 1)
        return carry

    lax.fori_loop(0, _NG // 2, body, 0)


_sc_gather = pl.kernel(
    _gather_body,
    out_type=jax.ShapeDtypeStruct((B * C * N,), jnp.float32),
    mesh=plsc.VectorSubcoreMesh(core_axis_name="c", subcore_axis_name="s"),
    scratch_types=[
        pltpu.VMEM((N,), jnp.int32),
        pltpu.VMEM((_CPG, _CHUNK), jnp.int32),
        pltpu.VMEM((_CPG, _CHUNK), jnp.int32),
        pltpu.VMEM((_GC * N,), jnp.float32),
        pltpu.VMEM((_GC * N,), jnp.float32),
        pltpu.SemaphoreType.DMA,
        pltpu.SemaphoreType.DMA,
    ],
)


@jax.jit
def kernel(xyz, features):
    x = xyz[:, :, 0]
    y = xyz[:, :, 1]
    z = xyz[:, :, 2]
    sample_inds, nxyz = _fps(x, y, z)
    out_flat = _sc_gather(features.reshape(-1), sample_inds.reshape(-1))
    new_features = out_flat.reshape(B, C, N)
    new_xyz = nxyz.reshape(B, N, 3)
    return new_xyz, new_features, sample_inds
